# Initial kernel scaffold; baseline (speedup 1.0000x reference)
import jax
import jax.numpy as jnp
from jax.experimental import pallas as pl


def kernel(x, edge_index, l1_w1, l1_b1, l1_w2, l1_w3, l1_b3, l2_w1, l2_b1, l2_w2, l2_w3, l2_b3):
    raise NotImplementedError("write your pallas kernel here")


# speedup vs baseline: 5.9904x; 5.9904x over previous
import functools

import jax
import jax.numpy as jnp
from jax import lax
from jax.experimental import pallas as pl
from jax.experimental.pallas import tpu as pltpu
from jax.experimental.pallas import tpu_sc as plsc

NC = 2
NS = 16
NW = NC * NS
CHUNK = 128
ZCHUNK = 128


def _sc_agg_body(epw, rows_per_tile, x_hbm, src_hbm, dst_hbm,
                 acc0_out, acc1_out, deg0_out, deg1_out,
                 src_v, dst_v, rows_v, ones_v, zrow_v, zcol_v,
                 acc_sh, deg_sh, sem):
    c = lax.axis_index("c")
    s = lax.axis_index("s")
    feat = x_hbm.shape[1]
    zeros16 = jnp.zeros((16,), jnp.float32)
    ones16 = jnp.ones((16,), jnp.float32)

    def fill_zrow(i, carry):
        for j in range(feat // 16):
            zrow_v[i, pl.ds(j * 16, 16)] = zeros16
        return carry
    lax.fori_loop(0, ZCHUNK, fill_zrow, None)

    def fill_zcol(k, carry):
        zcol_v[pl.ds(k * 16, 16)] = zeros16
        return carry
    lax.fori_loop(0, rows_per_tile // 16, fill_zcol, None)

    def fill_ones(k, carry):
        ones_v[pl.ds(k * 16, 16)] = ones16
        return carry
    lax.fori_loop(0, CHUNK // 16, fill_ones, None)

    row0 = s * rows_per_tile
    for k in range(rows_per_tile // ZCHUNK):
        pltpu.sync_copy(zrow_v, acc_sh.at[pl.ds(row0 + k * ZCHUNK, ZCHUNK), :])
    pltpu.sync_copy(zcol_v, deg_sh.at[pl.ds(row0, rows_per_tile)])
    plsc.subcore_barrier()

    w = c * NS + s
    base0 = w * epw

    def chunk_body(g, carry):
        base = base0 + g * CHUNK
        pltpu.sync_copy(src_hbm.at[pl.ds(base, CHUNK)], src_v)
        pltpu.sync_copy(dst_hbm.at[pl.ds(base, CHUNK)], dst_v)
        pltpu.async_copy(x_hbm.at[src_v], rows_v, sem).wait()
        pltpu.sync_copy(rows_v, acc_sh.at[dst_v], add=True)
        pltpu.sync_copy(ones_v, deg_sh.at[dst_v], add=True)
        return carry
    lax.fori_loop(0, epw // CHUNK, chunk_body, None)
    plsc.subcore_barrier()

    @pl.when(c == 0)
    def _():
        pltpu.sync_copy(acc_sh.at[pl.ds(row0, rows_per_tile), :],
                        acc0_out.at[pl.ds(row0, rows_per_tile), :])
        pltpu.sync_copy(deg_sh.at[pl.ds(row0, rows_per_tile)],
                        deg0_out.at[pl.ds(row0, rows_per_tile)])

    @pl.when(c == 1)
    def _():
        pltpu.sync_copy(acc_sh.at[pl.ds(row0, rows_per_tile), :],
                        acc1_out.at[pl.ds(row0, rows_per_tile), :])
        pltpu.sync_copy(deg_sh.at[pl.ds(row0, rows_per_tile)],
                        deg1_out.at[pl.ds(row0, rows_per_tile)])


def _make_sc_agg(n_acc, feat, epw):
    rows_per_tile = n_acc // NS
    mesh = plsc.VectorSubcoreMesh(core_axis_name="c", subcore_axis_name="s",
                                  num_cores=NC, num_subcores=NS)
    return pl.kernel(
        functools.partial(_sc_agg_body, epw, rows_per_tile),
        out_type=[
            jax.ShapeDtypeStruct((n_acc, feat), jnp.float32),
            jax.ShapeDtypeStruct((n_acc, feat), jnp.float32),
            jax.ShapeDtypeStruct((n_acc,), jnp.float32),
            jax.ShapeDtypeStruct((n_acc,), jnp.float32),
        ],
        mesh=mesh,
        scratch_types=[
            pltpu.VMEM((CHUNK,), jnp.int32),
            pltpu.VMEM((CHUNK,), jnp.int32),
            pltpu.VMEM((CHUNK, feat), jnp.float32),
            pltpu.VMEM((CHUNK,), jnp.float32),
            pltpu.VMEM((ZCHUNK, feat), jnp.float32),
            pltpu.VMEM((rows_per_tile,), jnp.float32),
            pltpu.VMEM_SHARED((n_acc, feat), jnp.float32),
            pltpu.VMEM_SHARED((n_acc,), jnp.float32),
            pltpu.SemaphoreType.DMA,
        ],
    )


def _tc_combine_body(do_relu, x_ref, a0_ref, a1_ref, d0_ref, d1_ref,
                     w1_ref, w2_ref, w3_ref, b1_ref, b3_ref, o_ref):
    f32 = jnp.float32
    agg = a0_ref[...] + a1_ref[...]
    xv = x_ref[...]
    deg = d0_ref[...] + d1_ref[...]
    out = jnp.dot(agg, w1_ref[...], preferred_element_type=f32)
    out = out + deg * (b1_ref[...] - jnp.dot(xv, w2_ref[...], preferred_element_type=f32))
    out = out + jnp.dot(xv, w3_ref[...], preferred_element_type=f32) + b3_ref[...]
    if do_relu:
        out = jnp.maximum(out, 0.0)
    o_ref[...] = out


def _tc_combine(x, a0, a1, d0, d1, w1, w2, w3, b1, b3, do_relu, blk=1000):
    n, feat = x.shape
    rowspec = pl.BlockSpec((blk, feat), lambda i: (i, 0))
    degspec = pl.BlockSpec((blk, 1), lambda i: (i, 0))
    wspec = pl.BlockSpec((feat, feat), lambda i: (0, 0))
    bspec = pl.BlockSpec((1, feat), lambda i: (0, 0))
    return pl.pallas_call(
        functools.partial(_tc_combine_body, do_relu),
        grid=(n // blk,),
        in_specs=[rowspec, rowspec, rowspec, degspec, degspec,
                  wspec, wspec, wspec, bspec, bspec],
        out_specs=rowspec,
        out_shape=jax.ShapeDtypeStruct((n, feat), jnp.float32),
    )(x, a0, a1, d0, d1, w1, w2, w3, b1, b3)


def kernel(x, edge_index, l1_w1, l1_b1, l1_w2, l1_w3, l1_b3,
           l2_w1, l2_b1, l2_w2, l2_w3, l2_b3):
    n, feat = x.shape
    e = edge_index.shape[1]
    e_pad = -(-e // (NW * CHUNK)) * (NW * CHUNK)
    epw = e_pad // NW
    n_acc = -(-(n + 1) // (NS * ZCHUNK)) * (NS * ZCHUNK)
    sink = n

    src = edge_index[0]
    dst = edge_index[1]
    if e_pad != e:
        src = jnp.concatenate([src, jnp.zeros((e_pad - e,), jnp.int32)])
        dst = jnp.concatenate([dst, jnp.full((e_pad - e,), sink, jnp.int32)])

    sc_agg = _make_sc_agg(n_acc, feat, epw)

    a0, a1, d0, d1 = sc_agg(x, src, dst)
    d0r = d0.reshape(n_acc, 1)
    d1r = d1.reshape(n_acc, 1)
    b1r = l1_b1.reshape(1, feat)
    b3r = l1_b3.reshape(1, feat)
    h = _tc_combine(x, a0, a1, d0r, d1r, l1_w1, l1_w2, l1_w3, b1r, b3r,
                    do_relu=True)

    g0, g1, _, _ = sc_agg(h, src, dst)
    out = _tc_combine(h, g0, g1, d0r, d1r, l2_w1, l2_w2, l2_w3,
                      l2_b1.reshape(1, feat), l2_b3.reshape(1, feat),
                      do_relu=False)
    return out
